# baseline (speedup 1.0000x reference)
import functools

import jax
import jax.numpy as jnp
from jax import lax
from jax.experimental import pallas as pl
from jax.experimental.pallas import tpu as pltpu
from jax.experimental.pallas import tpu_sc as plsc

NUM_USERS = 25000
NUM_ITEMS = 25000
NUM_NODES = NUM_USERS + NUM_ITEMS
EMBED_DIM = 64
NUM_EDGES = 800000
NUM_LAYERS = 3

HALF = 25000
ACC_ROWS = 25088
ROWS_PER_TILE = ACC_ROWS // 16
CHUNK = 128

E_PAD = 811008
P_EDGES = E_PAD // 32
P_CHUNK = 384
P_NCHUNK = P_EDGES // P_CHUNK
CAP_BLOCKS = 105
REGION = CAP_BLOCKS * CHUNK
LIST_LEN = 32 * REGION
NCHUNK = 2 * CAP_BLOCKS
DUMMY_DST = 2 * HALF


def _partition_edges(src, dst, w):
    mesh = plsc.VectorSubcoreMesh(core_axis_name="c", subcore_axis_name="s")

    @functools.partial(
        pl.kernel,
        mesh=mesh,
        compiler_params=pltpu.CompilerParams(needs_layout_passes=False,
                                             use_tc_tiling_on_sc=False),
        out_type=(jax.ShapeDtypeStruct((2 * LIST_LEN,), jnp.int32),
                  jax.ShapeDtypeStruct((2 * LIST_LEN,), jnp.int32),
                  jax.ShapeDtypeStruct((2 * LIST_LEN,), jnp.float32)),
        scratch_types=[pltpu.VMEM((P_CHUNK,), jnp.int32)] * 2
                      + [pltpu.VMEM((P_CHUNK,), jnp.int32)] * 2
                      + [pltpu.VMEM((P_CHUNK,), jnp.float32)] * 2
                      + [pltpu.VMEM((256,), jnp.int32),
                         pltpu.VMEM((256,), jnp.int32),
                         pltpu.VMEM((256,), jnp.float32),
                         pltpu.VMEM((256,), jnp.int32),
                         pltpu.VMEM((256,), jnp.int32),
                         pltpu.VMEM((256,), jnp.float32),
                         pltpu.VMEM((CHUNK,), jnp.int32),
                         pltpu.VMEM((CHUNK,), jnp.int32),
                         pltpu.VMEM((CHUNK,), jnp.float32),
                         pltpu.SemaphoreType.DMA,
                         pltpu.SemaphoreType.DMA],
    )
    def part(src_hbm, dst_hbm, w_hbm, psrc_hbm, pdst_hbm, pw_hbm, *scr):
        c = lax.axis_index("c")
        s = lax.axis_index("s")
        wid = c * 16 + s
        in_base = wid * P_EDGES
        rbase = (wid * REGION, LIST_LEN + wid * REGION)

        sbuf = scr[0:2]
        dbuf = scr[2:4]
        wbuf = scr[4:6]
        obuf = ((scr[6], scr[7], scr[8]), (scr[9], scr[10], scr[11]))
        dmy = (scr[12], scr[13], scr[14])
        esem = (scr[15], scr[16])

        iota = lax.iota(jnp.int32, 16)

        for k in range(CHUNK // 16):
            sl = pl.ds(k * 16, 16)
            dmy[0][sl] = iota * 0
            dmy[1][sl] = iota * 0 + DUMMY_DST
            dmy[2][sl] = (iota * 0).astype(jnp.float32)

        def start_edges(b, ic):
            eb = in_base + ic * P_CHUNK
            pltpu.async_copy(src_hbm.at[pl.ds(eb, P_CHUNK)], sbuf[b], esem[b])
            pltpu.async_copy(dst_hbm.at[pl.ds(eb, P_CHUNK)], dbuf[b], esem[b])
            pltpu.async_copy(w_hbm.at[pl.ds(eb, P_CHUNK)], wbuf[b], esem[b])

        def wait_edges(b, ic):
            eb = in_base + ic * P_CHUNK
            pltpu.make_async_copy(src_hbm.at[pl.ds(eb, P_CHUNK)], sbuf[b],
                                  esem[b]).wait()
            pltpu.make_async_copy(dst_hbm.at[pl.ds(eb, P_CHUNK)], dbuf[b],
                                  esem[b]).wait()
            pltpu.make_async_copy(w_hbm.at[pl.ds(eb, P_CHUNK)], wbuf[b],
                                  esem[b]).wait()

        out_hbms = (psrc_hbm, pdst_hbm, pw_hbm)

        def append_half(h, dest, vals, mask, cnt, blk):
            os_, od_, ow_ = obuf[h]
            plsc.store_scatter(os_, [dest], vals[0], mask=mask)
            plsc.store_scatter(od_, [dest], vals[1], mask=mask)
            plsc.store_scatter(ow_, [dest], vals[2], mask=mask)
            n = jnp.sum(mask.astype(jnp.int32))
            cnt = cnt + n
            flush = cnt >= CHUNK

            @pl.when(flush)
            def _flush():
                off = rbase[h] + jnp.minimum(blk, CAP_BLOCKS - 1) * CHUNK
                for f in range(3):
                    pltpu.sync_copy(obuf[h][f].at[pl.ds(0, CHUNK)],
                                    out_hbms[f].at[pl.ds(off, CHUNK)])
                for f in range(3):
                    obuf[h][f][pl.ds(0, 16)] = obuf[h][f][pl.ds(CHUNK, 16)]

            cnt = jnp.where(flush, cnt - CHUNK, cnt)
            blk = jnp.where(flush, blk + 1, blk)
            return cnt, blk

        def chunk_body(i, carry):
            cnt0, blk0, cnt1, blk1 = carry
            for b in (0, 1):
                ic = 2 * i + b
                wait_edges(b, ic)
                for g in range(P_CHUNK // 16):
                    sl = pl.ds(g * 16, 16)
                    s16 = sbuf[b][sl]
                    d16 = dbuf[b][sl]
                    w16 = wbuf[b][sl]
                    m0 = d16 < HALF
                    pos0 = plsc.cumsum(m0.astype(jnp.int32))
                    dest0 = cnt0 + pos0 - 1
                    cnt0, blk0 = append_half(0, dest0, (s16, d16, w16),
                                             m0, cnt0, blk0)
                    pos1 = (iota + 1) - pos0
                    dest1 = cnt1 + pos1 - 1
                    cnt1, blk1 = append_half(1, dest1, (s16, d16, w16),
                                             ~m0, cnt1, blk1)

                @pl.when(ic + 2 < P_NCHUNK)
                def _start():
                    start_edges(b, ic + 2)

            return cnt0, blk0, cnt1, blk1

        start_edges(0, 0)
        start_edges(1, 1)
        cnt0, blk0, cnt1, blk1 = lax.fori_loop(
            0, P_NCHUNK // 2, chunk_body,
            (jnp.int32(0), jnp.int32(0), jnp.int32(0), jnp.int32(0)))

        for h, cnt, blk in ((0, cnt0, blk0), (1, cnt1, blk1)):
            os_, od_, ow_ = obuf[h]
            for k in range(CHUNK // 16):
                destk = cnt + k * 16 + iota
                plsc.store_scatter(os_, [destk], iota * 0)
                plsc.store_scatter(od_, [destk], iota * 0 + DUMMY_DST)
                plsc.store_scatter(ow_, [destk],
                                   (iota * 0).astype(jnp.float32))
            off = rbase[h] + jnp.minimum(blk, CAP_BLOCKS - 1) * CHUNK
            for f in range(3):
                pltpu.sync_copy(obuf[h][f].at[pl.ds(0, CHUNK)],
                                out_hbms[f].at[pl.ds(off, CHUNK)])

            def fill_body(j, carry, h=h):
                off_j = rbase[h] + j * CHUNK
                for f in range(3):
                    pltpu.sync_copy(dmy[f],
                                    out_hbms[f].at[pl.ds(off_j, CHUNK)])
                return carry

            lax.fori_loop(jnp.minimum(blk + 1, CAP_BLOCKS), CAP_BLOCKS,
                          fill_body, 0)

    return part(src, dst, w)


def _propagate_layer(emb, psrc, pdst, pw, zeros_acc):
    mesh = plsc.VectorSubcoreMesh(core_axis_name="c", subcore_axis_name="s")

    @functools.partial(
        pl.kernel,
        mesh=mesh,
        compiler_params=pltpu.CompilerParams(needs_layout_passes=False,
                                             use_tc_tiling_on_sc=False),
        out_type=jax.ShapeDtypeStruct((NUM_NODES, EMBED_DIM), jnp.float32),
        scratch_types=[
            pltpu.VMEM_SHARED((ACC_ROWS, EMBED_DIM), jnp.float32),
        ] + [pltpu.VMEM((CHUNK,), jnp.int32)] * 2
          + [pltpu.VMEM((CHUNK,), jnp.int32)] * 2
          + [pltpu.VMEM((CHUNK,), jnp.float32)] * 2
          + [pltpu.VMEM((CHUNK,), jnp.int32)] * 2
          + [pltpu.VMEM((CHUNK, EMBED_DIM), jnp.float32)] * 2
          + [pltpu.SemaphoreType.DMA] * 4,
    )
    def layer(emb_hbm, src_hbm, dst_hbm, w_hbm, zeros_hbm, out_hbm, acc,
              *scr):
        c = lax.axis_index("c")
        s = lax.axis_index("s")
        sbuf = scr[0:2]
        dbuf = scr[2:4]
        wbuf = scr[4:6]
        dstl = scr[6:8]
        rows = scr[8:10]
        esem = scr[10:12]
        gsem = scr[12:14]

        node_base = c * HALF
        tile_edge_base = c * LIST_LEN + s * (2 * REGION)

        zr = s * ROWS_PER_TILE
        pltpu.sync_copy(zeros_hbm.at[pl.ds(zr, ROWS_PER_TILE)],
                        acc.at[pl.ds(zr, ROWS_PER_TILE)])
        plsc.subcore_barrier()

        def start_edges(b, ic):
            eb = tile_edge_base + ic * CHUNK
            pltpu.async_copy(src_hbm.at[pl.ds(eb, CHUNK)], sbuf[b], esem[b])
            pltpu.async_copy(dst_hbm.at[pl.ds(eb, CHUNK)], dbuf[b], esem[b])
            pltpu.async_copy(w_hbm.at[pl.ds(eb, CHUNK)], wbuf[b], esem[b])

        def wait_edges(b, ic):
            eb = tile_edge_base + ic * CHUNK
            pltpu.make_async_copy(src_hbm.at[pl.ds(eb, CHUNK)], sbuf[b],
                                  esem[b]).wait()
            pltpu.make_async_copy(dst_hbm.at[pl.ds(eb, CHUNK)], dbuf[b],
                                  esem[b]).wait()
            pltpu.make_async_copy(w_hbm.at[pl.ds(eb, CHUNK)], wbuf[b],
                                  esem[b]).wait()

        def localize(b):
            for g in range(CHUNK // 16):
                sl = pl.ds(g * 16, 16)
                dl = dbuf[b][sl] - node_base
                ok = (dl >= 0) & (dl < HALF)
                dstl[b][sl] = jnp.where(ok, dl, HALF)

        def scale(b):
            for g in range(CHUNK // 16):
                w16 = wbuf[b][pl.ds(g * 16, 16)]
                for e in range(16):
                    wspl = lax.gather(
                        w16,
                        jnp.full((16, 1), e, jnp.int32),
                        lax.GatherDimensionNumbers(
                            offset_dims=(), collapsed_slice_dims=(0,),
                            start_index_map=(0,)),
                        slice_sizes=(1,),
                        mode=lax.GatherScatterMode.PROMISE_IN_BOUNDS)
                    r = g * 16 + e
                    for cc in range(4):
                        sl = pl.ds(cc * 16, 16)
                        rows[b][r, sl] = rows[b][r, sl] * wspl

        for b in (0, 1):
            start_edges(b, b)
            wait_edges(b, b)
            pltpu.async_copy(emb_hbm.at[sbuf[b]], rows[b], gsem[b])

        def body(i, carry):
            for b in (0, 1):
                ic = 2 * i + b
                nxt = ic + 2

                pltpu.make_async_copy(emb_hbm.at[sbuf[b]], rows[b],
                                      gsem[b]).wait()
                localize(b)
                scale(b)

                @pl.when(nxt < NCHUNK)
                def _start():
                    start_edges(b, nxt)

                pltpu.sync_copy(rows[b], acc.at[dstl[b]], add=True)

                @pl.when(nxt < NCHUNK)
                def _refill():
                    wait_edges(b, nxt)
                    pltpu.async_copy(emb_hbm.at[sbuf[b]], rows[b], gsem[b])

            return carry

        lax.fori_loop(0, NCHUNK // 2, body, 0)

        plsc.subcore_barrier()

        fb = s * ROWS_PER_TILE
        ob = c * HALF + fb

        @pl.when(s < 15)
        def _flush_full():
            pltpu.sync_copy(acc.at[pl.ds(fb, ROWS_PER_TILE)],
                            out_hbm.at[pl.ds(ob, ROWS_PER_TILE)])

        @pl.when(s == 15)
        def _flush_tail():
            pltpu.sync_copy(acc.at[pl.ds(fb, HALF - 15 * ROWS_PER_TILE)],
                            out_hbm.at[pl.ds(ob, HALF - 15 * ROWS_PER_TILE)])

    return layer(emb, psrc, pdst, pw, zeros_acc)


def _mean4(e0, e1, e2, e3):
    a0 = e0.reshape(NUM_NODES // 2, 128)
    a1 = e1.reshape(NUM_NODES // 2, 128)
    a2 = e2.reshape(NUM_NODES // 2, 128)
    a3 = e3.reshape(NUM_NODES // 2, 128)

    def body(r0, r1, r2, r3, o):
        o[...] = (r0[...] + r1[...] + r2[...] + r3[...]) * 0.25

    spec = pl.BlockSpec((1000, 128), lambda i: (i, 0))
    out = pl.pallas_call(
        body,
        grid=(NUM_NODES // 2 // 1000,),
        in_specs=[spec, spec, spec, spec],
        out_specs=spec,
        out_shape=jax.ShapeDtypeStruct((NUM_NODES // 2, 128), jnp.float32),
    )(a0, a1, a2, a3)
    return out.reshape(NUM_NODES, EMBED_DIM)


def kernel(user_emb, item_emb, edge_weight, edge_index):
    e0 = jnp.concatenate([user_emb, item_emb], axis=0)

    pad = E_PAD - NUM_EDGES
    src = jnp.concatenate([edge_index[0], jnp.zeros((pad,), jnp.int32)])
    dst = jnp.concatenate([edge_index[1], jnp.zeros((pad,), jnp.int32)])
    w = jnp.concatenate([edge_weight, jnp.zeros((pad,), jnp.float32)])

    psrc, pdst, pw = _partition_edges(src, dst, w)

    zeros_acc = jnp.zeros((ACC_ROWS, EMBED_DIM), jnp.float32)

    e1 = _propagate_layer(e0, psrc, pdst, pw, zeros_acc)
    e2 = _propagate_layer(e1, psrc, pdst, pw, zeros_acc)
    e3 = _propagate_layer(e2, psrc, pdst, pw, zeros_acc)

    final = _mean4(e0, e1, e2, e3)
    return (final[:NUM_USERS], final[NUM_USERS:])


# speedup vs baseline: 2.0878x; 2.0878x over previous
import functools

import jax
import jax.numpy as jnp
from jax import lax
from jax.experimental import pallas as pl
from jax.experimental.pallas import tpu as pltpu
from jax.experimental.pallas import tpu_sc as plsc

NUM_USERS = 25000
NUM_ITEMS = 25000
NUM_NODES = NUM_USERS + NUM_ITEMS
EMBED_DIM = 64
NUM_EDGES = 800000
NUM_LAYERS = 3

HALF = 25000
ACC_ROWS = 25088
ROWS_PER_TILE = ACC_ROWS // 16
CHUNK = 128
NCHUNK = 392
EDGES_PER_TILE = CHUNK * NCHUNK
E_PAD = EDGES_PER_TILE * 16


def _propagate_layer(emb, src, dst, w, zeros_acc):
    mesh = plsc.VectorSubcoreMesh(core_axis_name="c", subcore_axis_name="s")

    @functools.partial(
        pl.kernel,
        mesh=mesh,
        compiler_params=pltpu.CompilerParams(needs_layout_passes=False,
                                             use_tc_tiling_on_sc=False),
        out_type=jax.ShapeDtypeStruct((NUM_NODES, EMBED_DIM), jnp.float32),
        scratch_types=[
            pltpu.VMEM_SHARED((ACC_ROWS, EMBED_DIM), jnp.float32),
        ] + [pltpu.VMEM((CHUNK,), jnp.int32)] * 2
          + [pltpu.VMEM((CHUNK,), jnp.int32)] * 2
          + [pltpu.VMEM((CHUNK,), jnp.float32)] * 2
          + [pltpu.VMEM((CHUNK,), jnp.int32)] * 2
          + [pltpu.VMEM((CHUNK, EMBED_DIM), jnp.float32)] * 2
          + [pltpu.SemaphoreType.DMA] * 4,
    )
    def layer(emb_hbm, src_hbm, dst_hbm, w_hbm, zeros_hbm, out_hbm, acc,
              *scr):
        c = lax.axis_index("c")
        s = lax.axis_index("s")
        sbuf = scr[0:2]
        dbuf = scr[2:4]
        wbuf = scr[4:6]
        dstl = scr[6:8]
        rows = scr[8:10]
        esem = scr[10:12]
        gsem = scr[12:14]

        node_base = c * HALF
        tile_edge_base = s * EDGES_PER_TILE

        zr = s * ROWS_PER_TILE
        pltpu.sync_copy(zeros_hbm.at[pl.ds(zr, ROWS_PER_TILE)],
                        acc.at[pl.ds(zr, ROWS_PER_TILE)])
        plsc.subcore_barrier()

        def start_edges(b, ic):
            eb = tile_edge_base + ic * CHUNK
            pltpu.async_copy(src_hbm.at[pl.ds(eb, CHUNK)], sbuf[b], esem[b])
            pltpu.async_copy(dst_hbm.at[pl.ds(eb, CHUNK)], dbuf[b], esem[b])
            pltpu.async_copy(w_hbm.at[pl.ds(eb, CHUNK)], wbuf[b], esem[b])

        def wait_edges(b, ic):
            eb = tile_edge_base + ic * CHUNK
            pltpu.make_async_copy(src_hbm.at[pl.ds(eb, CHUNK)], sbuf[b],
                                  esem[b]).wait()
            pltpu.make_async_copy(dst_hbm.at[pl.ds(eb, CHUNK)], dbuf[b],
                                  esem[b]).wait()
            pltpu.make_async_copy(w_hbm.at[pl.ds(eb, CHUNK)], wbuf[b],
                                  esem[b]).wait()

        def localize(b):
            for g in range(CHUNK // 16):
                sl = pl.ds(g * 16, 16)
                dl = dbuf[b][sl] - node_base
                ok = (dl >= 0) & (dl < HALF)
                dstl[b][sl] = jnp.where(ok, dl, HALF)

        def scale(b):
            for g in range(CHUNK // 16):
                w16 = wbuf[b][pl.ds(g * 16, 16)]
                for e in range(16):
                    wspl = lax.gather(
                        w16,
                        jnp.full((16, 1), e, jnp.int32),
                        lax.GatherDimensionNumbers(
                            offset_dims=(), collapsed_slice_dims=(0,),
                            start_index_map=(0,)),
                        slice_sizes=(1,),
                        mode=lax.GatherScatterMode.PROMISE_IN_BOUNDS)
                    r = g * 16 + e
                    for cc in range(4):
                        sl = pl.ds(cc * 16, 16)
                        rows[b][r, sl] = rows[b][r, sl] * wspl

        for b in (0, 1):
            start_edges(b, b)
            wait_edges(b, b)
            pltpu.async_copy(emb_hbm.at[sbuf[b]], rows[b], gsem[b])

        def body(i, carry):
            for b in (0, 1):
                ic = 2 * i + b
                nxt = ic + 2

                pltpu.make_async_copy(emb_hbm.at[sbuf[b]], rows[b],
                                      gsem[b]).wait()
                localize(b)
                scale(b)

                @pl.when(nxt < NCHUNK)
                def _start():
                    start_edges(b, nxt)

                pltpu.sync_copy(rows[b], acc.at[dstl[b]], add=True)

                @pl.when(nxt < NCHUNK)
                def _refill():
                    wait_edges(b, nxt)
                    pltpu.async_copy(emb_hbm.at[sbuf[b]], rows[b], gsem[b])

            return carry

        lax.fori_loop(0, NCHUNK // 2, body, 0)

        plsc.subcore_barrier()

        fb = s * ROWS_PER_TILE
        ob = c * HALF + fb

        @pl.when(s < 15)
        def _flush_full():
            pltpu.sync_copy(acc.at[pl.ds(fb, ROWS_PER_TILE)],
                            out_hbm.at[pl.ds(ob, ROWS_PER_TILE)])

        @pl.when(s == 15)
        def _flush_tail():
            pltpu.sync_copy(acc.at[pl.ds(fb, HALF - 15 * ROWS_PER_TILE)],
                            out_hbm.at[pl.ds(ob, HALF - 15 * ROWS_PER_TILE)])

    return layer(emb, src, dst, w, zeros_acc)


def _mean4(e0, e1, e2, e3):
    a0 = e0.reshape(NUM_NODES // 2, 128)
    a1 = e1.reshape(NUM_NODES // 2, 128)
    a2 = e2.reshape(NUM_NODES // 2, 128)
    a3 = e3.reshape(NUM_NODES // 2, 128)

    def body(r0, r1, r2, r3, o):
        o[...] = (r0[...] + r1[...] + r2[...] + r3[...]) * 0.25

    spec = pl.BlockSpec((1000, 128), lambda i: (i, 0))
    out = pl.pallas_call(
        body,
        grid=(NUM_NODES // 2 // 1000,),
        in_specs=[spec, spec, spec, spec],
        out_specs=spec,
        out_shape=jax.ShapeDtypeStruct((NUM_NODES // 2, 128), jnp.float32),
    )(a0, a1, a2, a3)
    return out.reshape(NUM_NODES, EMBED_DIM)


def kernel(user_emb, item_emb, edge_weight, edge_index):
    e0 = jnp.concatenate([user_emb, item_emb], axis=0)

    pad = E_PAD - NUM_EDGES
    src = jnp.concatenate([edge_index[0], jnp.zeros((pad,), jnp.int32)])
    dst = jnp.concatenate([edge_index[1], jnp.zeros((pad,), jnp.int32)])
    w = jnp.concatenate([edge_weight, jnp.zeros((pad,), jnp.float32)])

    zeros_acc = jnp.zeros((ACC_ROWS, EMBED_DIM), jnp.float32)

    e1 = _propagate_layer(e0, src, dst, w, zeros_acc)
    e2 = _propagate_layer(e1, src, dst, w, zeros_acc)
    e3 = _propagate_layer(e2, src, dst, w, zeros_acc)

    final = _mean4(e0, e1, e2, e3)
    return (final[:NUM_USERS], final[NUM_USERS:])


# speedup vs baseline: 2.1022x; 1.0069x over previous
import functools

import jax
import jax.numpy as jnp
from jax import lax
from jax.experimental import pallas as pl
from jax.experimental.pallas import tpu as pltpu
from jax.experimental.pallas import tpu_sc as plsc

NUM_USERS = 25000
NUM_ITEMS = 25000
NUM_NODES = NUM_USERS + NUM_ITEMS
EMBED_DIM = 64
NUM_EDGES = 800000
NUM_LAYERS = 3

HALF = 25000
ACC_ROWS = 25088
ROWS_PER_TILE = ACC_ROWS // 16
CHUNK = 128
NCHUNK = 392
EDGES_PER_TILE = CHUNK * NCHUNK
E_PAD = EDGES_PER_TILE * 16


def _propagate_layer(emb, src, dst, w, zeros_acc):
    mesh = plsc.VectorSubcoreMesh(core_axis_name="c", subcore_axis_name="s")

    @functools.partial(
        pl.kernel,
        mesh=mesh,
        compiler_params=pltpu.CompilerParams(needs_layout_passes=False,
                                             use_tc_tiling_on_sc=False),
        out_type=jax.ShapeDtypeStruct((NUM_NODES, EMBED_DIM), jnp.float32),
        scratch_types=[
            pltpu.VMEM_SHARED((ACC_ROWS, EMBED_DIM), jnp.float32),
        ] + [pltpu.VMEM((CHUNK,), jnp.int32)] * 2
          + [pltpu.VMEM((CHUNK,), jnp.int32)] * 2
          + [pltpu.VMEM((CHUNK,), jnp.float32)] * 2
          + [pltpu.VMEM((CHUNK // 2,), jnp.int32)] * 4
          + [pltpu.VMEM((CHUNK, EMBED_DIM), jnp.float32)] * 2
          + [pltpu.SemaphoreType.DMA] * 6,
    )
    def layer(emb_hbm, src_hbm, dst_hbm, w_hbm, zeros_hbm, out_hbm, acc,
              *scr):
        c = lax.axis_index("c")
        s = lax.axis_index("s")
        sbuf = scr[0:2]
        dbuf = scr[2:4]
        wbuf = scr[4:6]
        dstlA = scr[6:8]
        dstlB = scr[8:10]
        rows = scr[10:12]
        esem = scr[12:14]
        gsem = scr[14:16]
        ssem = scr[16:18]

        node_base = c * HALF
        tile_edge_base = s * EDGES_PER_TILE

        zr = s * ROWS_PER_TILE
        pltpu.sync_copy(zeros_hbm.at[pl.ds(zr, ROWS_PER_TILE)],
                        acc.at[pl.ds(zr, ROWS_PER_TILE)])
        plsc.subcore_barrier()

        def start_edges(b, ic):
            eb = tile_edge_base + ic * CHUNK
            pltpu.async_copy(src_hbm.at[pl.ds(eb, CHUNK)], sbuf[b], esem[b])
            pltpu.async_copy(dst_hbm.at[pl.ds(eb, CHUNK)], dbuf[b], esem[b])
            pltpu.async_copy(w_hbm.at[pl.ds(eb, CHUNK)], wbuf[b], esem[b])

        def wait_edges(b, ic):
            eb = tile_edge_base + ic * CHUNK
            pltpu.make_async_copy(src_hbm.at[pl.ds(eb, CHUNK)], sbuf[b],
                                  esem[b]).wait()
            pltpu.make_async_copy(dst_hbm.at[pl.ds(eb, CHUNK)], dbuf[b],
                                  esem[b]).wait()
            pltpu.make_async_copy(w_hbm.at[pl.ds(eb, CHUNK)], wbuf[b],
                                  esem[b]).wait()

        def localize(b):
            for g in range(CHUNK // 16):
                sl = pl.ds(g * 16, 16)
                dl = dbuf[b][sl] - node_base
                ok = (dl >= 0) & (dl < HALF)
                tgt = dstlA[b] if g < CHUNK // 32 else dstlB[b]
                tgt[pl.ds((g % (CHUNK // 32)) * 16, 16)] = \
                    jnp.where(ok, dl, HALF)

        def scale(b, lo, hi):
            for g in range(lo, hi):
                w16 = wbuf[b][pl.ds(g * 16, 16)]
                for e in range(16):
                    wspl = lax.gather(
                        w16,
                        jnp.full((16, 1), e, jnp.int32),
                        lax.GatherDimensionNumbers(
                            offset_dims=(), collapsed_slice_dims=(0,),
                            start_index_map=(0,)),
                        slice_sizes=(1,),
                        mode=lax.GatherScatterMode.PROMISE_IN_BOUNDS)
                    r = g * 16 + e
                    for cc in range(4):
                        sl = pl.ds(cc * 16, 16)
                        rows[b][r, sl] = rows[b][r, sl] * wspl

        for b in (0, 1):
            start_edges(b, b)
            wait_edges(b, b)
            pltpu.async_copy(emb_hbm.at[sbuf[b]], rows[b], gsem[b])

        def body(i, carry):
            for b in (0, 1):
                ic = 2 * i + b
                nxt = ic + 2

                pltpu.make_async_copy(emb_hbm.at[sbuf[b]], rows[b],
                                      gsem[b]).wait()
                localize(b)
                scale(b, 0, CHUNK // 32)
                pltpu.async_copy(rows[b].at[pl.ds(0, CHUNK // 2)],
                                 acc.at[dstlA[b]], ssem[b], add=True)
                scale(b, CHUNK // 32, CHUNK // 16)

                @pl.when(nxt < NCHUNK)
                def _start():
                    start_edges(b, nxt)

                pltpu.sync_copy(rows[b].at[pl.ds(CHUNK // 2, CHUNK // 2)],
                                acc.at[dstlB[b]], add=True)
                pltpu.make_async_copy(rows[b].at[pl.ds(0, CHUNK // 2)],
                                      acc.at[dstlA[b]], ssem[b]).wait()

                @pl.when(nxt < NCHUNK)
                def _refill():
                    wait_edges(b, nxt)
                    pltpu.async_copy(emb_hbm.at[sbuf[b]], rows[b], gsem[b])

            return carry

        lax.fori_loop(0, NCHUNK // 2, body, 0)

        plsc.subcore_barrier()

        fb = s * ROWS_PER_TILE
        ob = c * HALF + fb

        @pl.when(s < 15)
        def _flush_full():
            pltpu.sync_copy(acc.at[pl.ds(fb, ROWS_PER_TILE)],
                            out_hbm.at[pl.ds(ob, ROWS_PER_TILE)])

        @pl.when(s == 15)
        def _flush_tail():
            pltpu.sync_copy(acc.at[pl.ds(fb, HALF - 15 * ROWS_PER_TILE)],
                            out_hbm.at[pl.ds(ob, HALF - 15 * ROWS_PER_TILE)])

    return layer(emb, src, dst, w, zeros_acc)


def _mean4(e0, e1, e2, e3):
    a0 = e0.reshape(NUM_NODES // 2, 128)
    a1 = e1.reshape(NUM_NODES // 2, 128)
    a2 = e2.reshape(NUM_NODES // 2, 128)
    a3 = e3.reshape(NUM_NODES // 2, 128)

    def body(r0, r1, r2, r3, o):
        o[...] = (r0[...] + r1[...] + r2[...] + r3[...]) * 0.25

    spec = pl.BlockSpec((1000, 128), lambda i: (i, 0))
    out = pl.pallas_call(
        body,
        grid=(NUM_NODES // 2 // 1000,),
        in_specs=[spec, spec, spec, spec],
        out_specs=spec,
        out_shape=jax.ShapeDtypeStruct((NUM_NODES // 2, 128), jnp.float32),
    )(a0, a1, a2, a3)
    return out.reshape(NUM_NODES, EMBED_DIM)


def kernel(user_emb, item_emb, edge_weight, edge_index):
    e0 = jnp.concatenate([user_emb, item_emb], axis=0)

    pad = E_PAD - NUM_EDGES
    src = jnp.concatenate([edge_index[0], jnp.zeros((pad,), jnp.int32)])
    dst = jnp.concatenate([edge_index[1], jnp.zeros((pad,), jnp.int32)])
    w = jnp.concatenate([edge_weight, jnp.zeros((pad,), jnp.float32)])

    zeros_acc = jnp.zeros((ACC_ROWS, EMBED_DIM), jnp.float32)

    e1 = _propagate_layer(e0, src, dst, w, zeros_acc)
    e2 = _propagate_layer(e1, src, dst, w, zeros_acc)
    e3 = _propagate_layer(e2, src, dst, w, zeros_acc)

    final = _mean4(e0, e1, e2, e3)
    return (final[:NUM_USERS], final[NUM_USERS:])
